# baseline (speedup 1.0000x reference)
import functools

import jax
import jax.numpy as jnp
from jax import lax
from jax.experimental import pallas as pl
from jax.experimental.pallas import tpu as pltpu
from jax.experimental.pallas import tpu_sc as plsc

NC = 2
NS = 16
NW = NC * NS


def _tc1_body(nf_ref, wa_ref, wb_ref, a_ref, b_ref):
    x = nf_ref[0]
    a_ref[0] = jnp.dot(x, wa_ref[...], preferred_element_type=jnp.float32)
    b_ref[0] = jnp.dot(x, wb_ref[...], preferred_element_type=jnp.float32)


def _tc1(nf, Wa, Wb, bn):
    B, N, D = nf.shape
    H = Wa.shape[1]
    grid = (B, N // bn)
    return pl.pallas_call(
        _tc1_body,
        grid=grid,
        in_specs=[
            pl.BlockSpec((1, bn, D), lambda b, i: (b, i, 0)),
            pl.BlockSpec((D, H), lambda b, i: (0, 0)),
            pl.BlockSpec((D, H), lambda b, i: (0, 0)),
        ],
        out_specs=[
            pl.BlockSpec((1, bn, H), lambda b, i: (b, i, 0)),
            pl.BlockSpec((1, bn, H), lambda b, i: (b, i, 0)),
        ],
        out_shape=[
            jax.ShapeDtypeStruct((B, N, H), jnp.float32),
            jax.ShapeDtypeStruct((B, N, H), jnp.float32),
        ],
    )(nf, Wa, Wb)


def _sc1_body(K, EPW, H, a_hbm, b_hbm, es_hbm, ed_hbm, out_hbm,
              idxa, idxb, bufa0, bufb0, bufa1, bufb1,
              sa0, sb0, sa1, sb1):
    c = lax.axis_index("c")
    s = lax.axis_index("s")
    wid = s * NC + c
    ebase = wid * EPW
    C = EPW // K
    bufa = [bufa0, bufa1]
    bufb = [bufb0, bufb1]
    sa = [sa0, sa1]
    sb = [sb0, sb1]

    pltpu.sync_copy(es_hbm.at[pl.ds(ebase, EPW)], idxa)
    pltpu.sync_copy(ed_hbm.at[pl.ds(ebase, EPW)], idxb)

    def start(i, d):
        off = i * K
        pltpu.async_copy(a_hbm.at[idxa.at[pl.ds(off, K)]], bufa[d], sa[d])
        pltpu.async_copy(b_hbm.at[idxb.at[pl.ds(off, K)]], bufb[d], sb[d])

    def finish(i, d):
        off = i * K
        pltpu.make_async_copy(a_hbm.at[idxa.at[pl.ds(off, K)]], bufa[d],
                              sa[d]).wait()
        pltpu.make_async_copy(b_hbm.at[idxb.at[pl.ds(off, K)]], bufb[d],
                              sb[d]).wait()

        def row(r, cc):
            for j in range(H // 16):
                sl = pl.ds(j * 16, 16)
                plsc.addupdate(bufa[d].at[r, sl], bufb[d][r, sl])
            return cc

        lax.fori_loop(0, K, row, 0)
        pltpu.sync_copy(bufa[d], out_hbm.at[pl.ds(ebase + i * K, K)])

    start(0, 0)

    def pair(g, cc):
        i = g * 2
        start(i + 1, 1)
        finish(i, 0)

        @pl.when(i + 2 < C)
        def _():
            start(i + 2, 0)

        finish(i + 1, 1)
        return cc

    lax.fori_loop(0, C // 2, pair, 0)
    if C % 2 == 1:
        finish(C - 1, 0)


def _sc1(a_flat, b_flat, es_flat, ed_flat, K=40):
    BE = es_flat.shape[0]
    H = a_flat.shape[1]
    EPW = BE // NW
    mesh = plsc.VectorSubcoreMesh(core_axis_name="c", subcore_axis_name="s")
    fn = pl.kernel(
        functools.partial(_sc1_body, K, EPW, H),
        out_type=jax.ShapeDtypeStruct((BE, H), jnp.float32),
        mesh=mesh,
        scratch_types=(
            [pltpu.VMEM((EPW,), jnp.int32)] * 2
            + [pltpu.VMEM((K, H), jnp.float32)] * 4
            + [pltpu.SemaphoreType.DMA] * 4
        ),
    )
    return fn(a_flat, b_flat, es_flat, ed_flat)


def _tc2_body(hs_ref, ef_ref, wc_ref, bm1_ref, wm2_ref, bm2_ref, msg_ref):
    h = hs_ref[0] + jnp.dot(ef_ref[0], wc_ref[...],
                            preferred_element_type=jnp.float32) + bm1_ref[...]
    h = jnp.maximum(h, 0.0)
    m = jnp.dot(h, wm2_ref[...], preferred_element_type=jnp.float32) + bm2_ref[...]
    msg_ref[0] = jnp.maximum(m, 0.0)


def _tc2(hsum, ef, Wc, bm1, Wm2, bm2, be):
    B, E, H = hsum.shape
    DE = ef.shape[2]
    grid = (B, E // be)
    return pl.pallas_call(
        _tc2_body,
        grid=grid,
        in_specs=[
            pl.BlockSpec((1, be, H), lambda b, i: (b, i, 0)),
            pl.BlockSpec((1, be, DE), lambda b, i: (b, i, 0)),
            pl.BlockSpec((DE, H), lambda b, i: (0, 0)),
            pl.BlockSpec((1, H), lambda b, i: (0, 0)),
            pl.BlockSpec((H, H), lambda b, i: (0, 0)),
            pl.BlockSpec((1, H), lambda b, i: (0, 0)),
        ],
        out_specs=pl.BlockSpec((1, be, H), lambda b, i: (b, i, 0)),
        out_shape=jax.ShapeDtypeStruct((B, E, H), jnp.float32),
    )(hsum, ef, Wc, bm1, Wm2, bm2)


def _sc2_body(B, E, N, H, K, msg_hbm, edr_hbm, agg_hbm,
              idx, buf0, buf1, shared, sm0, sm1):
    HC = H // NC
    RZ = K
    NCH = N // RZ
    NT = (NCH + NS - 1) // NS
    EPT = E // NS
    CB = EPT // K
    c = lax.axis_index("c")
    s = lax.axis_index("s")
    col0 = c * HC
    buf = [buf0, buf1]
    sm = [sm0, sm1]

    def zrow(r, cc):
        for j in range(HC // 16):
            buf0[r, pl.ds(j * 16, 16)] = jnp.zeros((16,), jnp.float32)
        return cc

    def start(b, i, d):
        base = s * EPT + i * K
        pltpu.async_copy(msg_hbm.at[b, pl.ds(base, K), pl.ds(col0, HC)],
                         buf[d], sm[d])

    def finish(b, i, d):
        base = s * EPT + i * K
        pltpu.make_async_copy(msg_hbm.at[b, pl.ds(base, K), pl.ds(col0, HC)],
                              buf[d], sm[d]).wait()
        pltpu.sync_copy(buf[d], shared.at[idx.at[pl.ds(i * K, K)]], add=True)

    for b in range(B):
        pltpu.sync_copy(edr_hbm.at[pl.ds(b * E + s * EPT, EPT)], idx)
        lax.fori_loop(0, RZ, zrow, 0)
        for t in range(NT):
            j = s + t * NS

            @pl.when(j < NCH)
            def _():
                pltpu.sync_copy(buf0, shared.at[pl.ds(j * RZ, RZ)])

        plsc.subcore_barrier()

        start(b, 0, 0)

        def pair(g, cc):
            i = g * 2
            start(b, i + 1, 1)
            finish(b, i, 0)

            @pl.when(i + 2 < CB)
            def _():
                start(b, i + 2, 0)

            finish(b, i + 1, 1)
            return cc

        lax.fori_loop(0, CB // 2, pair, 0)
        plsc.subcore_barrier()

        for t in range(NT):
            j = s + t * NS

            @pl.when(j < NCH)
            def _():
                r0 = pl.multiple_of(j * RZ, RZ)
                pltpu.sync_copy(shared.at[pl.ds(r0, RZ)], buf0)
                pltpu.sync_copy(buf0, agg_hbm.at[b, pl.ds(r0, RZ), pl.ds(col0, HC)])

        plsc.subcore_barrier()


def _sc2(msg, ed_raw_flat, N, K=40):
    B, E, H = msg.shape
    EPT = E // NS
    mesh = plsc.VectorSubcoreMesh(core_axis_name="c", subcore_axis_name="s")
    fn = pl.kernel(
        functools.partial(_sc2_body, B, E, N, H, K),
        out_type=jax.ShapeDtypeStruct((B, N, H), jnp.float32),
        mesh=mesh,
        scratch_types=[
            pltpu.VMEM((EPT,), jnp.int32),
            pltpu.VMEM((K, H // NC), jnp.float32),
            pltpu.VMEM((K, H // NC), jnp.float32),
            pltpu.VMEM_SHARED((N, H // NC), jnp.float32),
            pltpu.SemaphoreType.DMA,
            pltpu.SemaphoreType.DMA,
        ],
    )
    return fn(msg, ed_raw_flat)


def _tc3_body(nf_ref, agg_ref, wua_ref, wub_ref, bu1_ref, wu2_ref, bu2_ref, out_ref):
    h = (jnp.dot(nf_ref[0], wua_ref[...], preferred_element_type=jnp.float32)
         + jnp.dot(agg_ref[0], wub_ref[...], preferred_element_type=jnp.float32)
         + bu1_ref[...])
    h = jnp.maximum(h, 0.0)
    o = jnp.dot(h, wu2_ref[...], preferred_element_type=jnp.float32) + bu2_ref[...]
    out_ref[0] = jnp.maximum(o, 0.0)


def _tc3(nf, agg, Wua, Wub, bu1, Wu2, bu2, bn):
    B, N, D = nf.shape
    H = agg.shape[2]
    grid = (B, N // bn)
    return pl.pallas_call(
        _tc3_body,
        grid=grid,
        in_specs=[
            pl.BlockSpec((1, bn, D), lambda b, i: (b, i, 0)),
            pl.BlockSpec((1, bn, H), lambda b, i: (b, i, 0)),
            pl.BlockSpec((D, H), lambda b, i: (0, 0)),
            pl.BlockSpec((H, H), lambda b, i: (0, 0)),
            pl.BlockSpec((1, H), lambda b, i: (0, 0)),
            pl.BlockSpec((H, H), lambda b, i: (0, 0)),
            pl.BlockSpec((1, H), lambda b, i: (0, 0)),
        ],
        out_specs=pl.BlockSpec((1, bn, H), lambda b, i: (b, i, 0)),
        out_shape=jax.ShapeDtypeStruct((B, N, H), jnp.float32),
    )(nf, agg, Wua, Wub, bu1, Wu2, bu2)


def kernel(node_features, edge_features, edge_src, edge_dst,
           Wm1, bm1, Wm2, bm2, Wu1, bu1, Wu2, bu2):
    B, N, D = node_features.shape
    E = edge_src.shape[1]
    H = Wm2.shape[0]

    Wa = Wm1[:D]
    Wb = Wm1[D:2 * D]
    Wc = Wm1[2 * D:]
    Wua = Wu1[:D]
    Wub = Wu1[D:]

    G = 4
    BG = B // G
    offs = (jnp.arange(BG, dtype=jnp.int32) * N)[:, None]
    outs = []
    for g in range(G):
        bsl = slice(g * BG, (g + 1) * BG)
        nf_g = node_features[bsl]
        A_g, Bm_g = _tc1(nf_g, Wa, Wb, bn=2000)
        es_g = (edge_src[bsl] + offs).reshape(BG * E)
        ed_g = (edge_dst[bsl] + offs).reshape(BG * E)
        hsum_g = _sc1(A_g.reshape(BG * N, H), Bm_g.reshape(BG * N, H),
                      es_g, ed_g)
        msg_g = _tc2(hsum_g.reshape(BG, E, H), edge_features[bsl], Wc,
                     bm1.reshape(1, H), Wm2, bm2.reshape(1, H), be=2000)
        agg_g = _sc2(msg_g, edge_dst[bsl].reshape(BG * E), N)
        outs.append(_tc3(nf_g, agg_g, Wua, Wub,
                         bu1.reshape(1, H), Wu2, bu2.reshape(1, H), bn=2000))
    return jnp.concatenate(outs, axis=0)


# speedup vs baseline: 1.0344x; 1.0344x over previous
import functools

import jax
import jax.numpy as jnp
from jax import lax
from jax.experimental import pallas as pl
from jax.experimental.pallas import tpu as pltpu
from jax.experimental.pallas import tpu_sc as plsc

NC = 2
NS = 16
NW = NC * NS


def _tc1_body(nf_ref, wa_ref, wb_ref, a_ref, b_ref):
    x = nf_ref[0]
    a_ref[0] = jnp.dot(x, wa_ref[...], preferred_element_type=jnp.float32)
    b_ref[0] = jnp.dot(x, wb_ref[...], preferred_element_type=jnp.float32)


def _tc1(nf, Wa, Wb, bn):
    B, N, D = nf.shape
    H = Wa.shape[1]
    grid = (B, N // bn)
    return pl.pallas_call(
        _tc1_body,
        grid=grid,
        in_specs=[
            pl.BlockSpec((1, bn, D), lambda b, i: (b, i, 0)),
            pl.BlockSpec((D, H), lambda b, i: (0, 0)),
            pl.BlockSpec((D, H), lambda b, i: (0, 0)),
        ],
        out_specs=[
            pl.BlockSpec((1, bn, H), lambda b, i: (b, i, 0)),
            pl.BlockSpec((1, bn, H), lambda b, i: (b, i, 0)),
        ],
        out_shape=[
            jax.ShapeDtypeStruct((B, N, H), jnp.float32),
            jax.ShapeDtypeStruct((B, N, H), jnp.float32),
        ],
    )(nf, Wa, Wb)


def _sc1_body(K, EPW, H, a_hbm, b_hbm, es_hbm, ed_hbm, out_hbm,
              idxa, idxb, bufa0, bufb0, bufa1, bufb1,
              sa0, sb0, sa1, sb1):
    c = lax.axis_index("c")
    s = lax.axis_index("s")
    wid = s * NC + c
    ebase = wid * EPW
    C = EPW // K
    bufa = [bufa0, bufa1]
    bufb = [bufb0, bufb1]
    sa = [sa0, sa1]
    sb = [sb0, sb1]

    pltpu.sync_copy(es_hbm.at[pl.ds(ebase, EPW)], idxa)
    pltpu.sync_copy(ed_hbm.at[pl.ds(ebase, EPW)], idxb)

    def start(i, d):
        off = i * K
        pltpu.async_copy(a_hbm.at[idxa.at[pl.ds(off, K)]], bufa[d], sa[d])
        pltpu.async_copy(b_hbm.at[idxb.at[pl.ds(off, K)]], bufb[d], sb[d])

    def finish(i, d):
        off = i * K
        pltpu.make_async_copy(a_hbm.at[idxa.at[pl.ds(off, K)]], bufa[d],
                              sa[d]).wait()
        pltpu.make_async_copy(b_hbm.at[idxb.at[pl.ds(off, K)]], bufb[d],
                              sb[d]).wait()

        def row(r, cc):
            for j in range(H // 16):
                sl = pl.ds(j * 16, 16)
                plsc.addupdate(bufa[d].at[r, sl], bufb[d][r, sl])
            return cc

        lax.fori_loop(0, K, row, 0)
        pltpu.sync_copy(bufa[d], out_hbm.at[pl.ds(ebase + i * K, K)])

    start(0, 0)

    def pair(g, cc):
        i = g * 2
        start(i + 1, 1)
        finish(i, 0)

        @pl.when(i + 2 < C)
        def _():
            start(i + 2, 0)

        finish(i + 1, 1)
        return cc

    lax.fori_loop(0, C // 2, pair, 0)
    if C % 2 == 1:
        finish(C - 1, 0)


def _sc1(a_flat, b_flat, es_flat, ed_flat, K=40):
    BE = es_flat.shape[0]
    H = a_flat.shape[1]
    EPW = BE // NW
    mesh = plsc.VectorSubcoreMesh(core_axis_name="c", subcore_axis_name="s")
    fn = pl.kernel(
        functools.partial(_sc1_body, K, EPW, H),
        out_type=jax.ShapeDtypeStruct((BE, H), jnp.float32),
        mesh=mesh,
        scratch_types=(
            [pltpu.VMEM((EPW,), jnp.int32)] * 2
            + [pltpu.VMEM((K, H), jnp.float32)] * 4
            + [pltpu.SemaphoreType.DMA] * 4
        ),
    )
    return fn(a_flat, b_flat, es_flat, ed_flat)


def _tc2_body(hs_ref, ef_ref, wc_ref, bm1_ref, wm2_ref, bm2_ref, msg_ref):
    h = hs_ref[0] + jnp.dot(ef_ref[0], wc_ref[...],
                            preferred_element_type=jnp.float32) + bm1_ref[...]
    h = jnp.maximum(h, 0.0)
    m = jnp.dot(h, wm2_ref[...], preferred_element_type=jnp.float32) + bm2_ref[...]
    msg_ref[0] = jnp.maximum(m, 0.0)


def _tc2(hsum, ef, Wc, bm1, Wm2, bm2, be):
    B, E, H = hsum.shape
    DE = ef.shape[2]
    grid = (B, E // be)
    return pl.pallas_call(
        _tc2_body,
        grid=grid,
        in_specs=[
            pl.BlockSpec((1, be, H), lambda b, i: (b, i, 0)),
            pl.BlockSpec((1, be, DE), lambda b, i: (b, i, 0)),
            pl.BlockSpec((DE, H), lambda b, i: (0, 0)),
            pl.BlockSpec((1, H), lambda b, i: (0, 0)),
            pl.BlockSpec((H, H), lambda b, i: (0, 0)),
            pl.BlockSpec((1, H), lambda b, i: (0, 0)),
        ],
        out_specs=pl.BlockSpec((1, be, H), lambda b, i: (b, i, 0)),
        out_shape=jax.ShapeDtypeStruct((B, E, H), jnp.float32),
    )(hsum, ef, Wc, bm1, Wm2, bm2)


def _sc2_body(B, E, N, H, K, msg_hbm, edr_hbm, agg_hbm,
              idx, buf0, buf1, shared, sm0, sm1):
    HC = H // NC
    RZ = K
    NCH = N // RZ
    NT = (NCH + NS - 1) // NS
    EPT = E // NS
    CB = EPT // K
    c = lax.axis_index("c")
    s = lax.axis_index("s")
    col0 = c * HC
    buf = [buf0, buf1]
    sm = [sm0, sm1]

    def zrow(r, cc):
        for j in range(HC // 16):
            buf0[r, pl.ds(j * 16, 16)] = jnp.zeros((16,), jnp.float32)
        return cc

    def start(b, i, d):
        base = s * EPT + i * K
        pltpu.async_copy(msg_hbm.at[b, pl.ds(base, K), pl.ds(col0, HC)],
                         buf[d], sm[d])

    def finish(b, i, d):
        base = s * EPT + i * K
        pltpu.make_async_copy(msg_hbm.at[b, pl.ds(base, K), pl.ds(col0, HC)],
                              buf[d], sm[d]).wait()
        pltpu.sync_copy(buf[d], shared.at[idx.at[pl.ds(i * K, K)]], add=True)

    for b in range(B):
        pltpu.sync_copy(edr_hbm.at[pl.ds(b * E + s * EPT, EPT)], idx)
        lax.fori_loop(0, RZ, zrow, 0)
        for t in range(NT):
            j = s + t * NS

            @pl.when(j < NCH)
            def _():
                pltpu.sync_copy(buf0, shared.at[pl.ds(j * RZ, RZ)])

        plsc.subcore_barrier()

        start(b, 0, 0)

        def pair(g, cc):
            i = g * 2
            start(b, i + 1, 1)
            finish(b, i, 0)

            @pl.when(i + 2 < CB)
            def _():
                start(b, i + 2, 0)

            finish(b, i + 1, 1)
            return cc

        lax.fori_loop(0, CB // 2, pair, 0)
        plsc.subcore_barrier()

        for t in range(NT):
            j = s + t * NS

            @pl.when(j < NCH)
            def _():
                r0 = pl.multiple_of(j * RZ, RZ)
                pltpu.sync_copy(shared.at[pl.ds(r0, RZ)], buf0)
                pltpu.sync_copy(buf0, agg_hbm.at[b, pl.ds(r0, RZ), pl.ds(col0, HC)])

        plsc.subcore_barrier()


def _sc2(msg, ed_raw_flat, N, K=40):
    B, E, H = msg.shape
    EPT = E // NS
    mesh = plsc.VectorSubcoreMesh(core_axis_name="c", subcore_axis_name="s")
    fn = pl.kernel(
        functools.partial(_sc2_body, B, E, N, H, K),
        out_type=jax.ShapeDtypeStruct((B, N, H), jnp.float32),
        mesh=mesh,
        scratch_types=[
            pltpu.VMEM((EPT,), jnp.int32),
            pltpu.VMEM((K, H // NC), jnp.float32),
            pltpu.VMEM((K, H // NC), jnp.float32),
            pltpu.VMEM_SHARED((N, H // NC), jnp.float32),
            pltpu.SemaphoreType.DMA,
            pltpu.SemaphoreType.DMA,
        ],
    )
    return fn(msg, ed_raw_flat)


def _tc3_body(nf_ref, agg_ref, wua_ref, wub_ref, bu1_ref, wu2_ref, bu2_ref, out_ref):
    h = (jnp.dot(nf_ref[0], wua_ref[...], preferred_element_type=jnp.float32)
         + jnp.dot(agg_ref[0], wub_ref[...], preferred_element_type=jnp.float32)
         + bu1_ref[...])
    h = jnp.maximum(h, 0.0)
    o = jnp.dot(h, wu2_ref[...], preferred_element_type=jnp.float32) + bu2_ref[...]
    out_ref[0] = jnp.maximum(o, 0.0)


def _tc3(nf, agg, Wua, Wub, bu1, Wu2, bu2, bn):
    B, N, D = nf.shape
    H = agg.shape[2]
    grid = (B, N // bn)
    return pl.pallas_call(
        _tc3_body,
        grid=grid,
        in_specs=[
            pl.BlockSpec((1, bn, D), lambda b, i: (b, i, 0)),
            pl.BlockSpec((1, bn, H), lambda b, i: (b, i, 0)),
            pl.BlockSpec((D, H), lambda b, i: (0, 0)),
            pl.BlockSpec((H, H), lambda b, i: (0, 0)),
            pl.BlockSpec((1, H), lambda b, i: (0, 0)),
            pl.BlockSpec((H, H), lambda b, i: (0, 0)),
            pl.BlockSpec((1, H), lambda b, i: (0, 0)),
        ],
        out_specs=pl.BlockSpec((1, bn, H), lambda b, i: (b, i, 0)),
        out_shape=jax.ShapeDtypeStruct((B, N, H), jnp.float32),
    )(nf, agg, Wua, Wub, bu1, Wu2, bu2)


def kernel(node_features, edge_features, edge_src, edge_dst,
           Wm1, bm1, Wm2, bm2, Wu1, bu1, Wu2, bu2):
    B, N, D = node_features.shape
    E = edge_src.shape[1]
    H = Wm2.shape[0]

    Wa = Wm1[:D]
    Wb = Wm1[D:2 * D]
    Wc = Wm1[2 * D:]
    Wua = Wu1[:D]
    Wub = Wu1[D:]

    G = 2
    BG = B // G
    offs = (jnp.arange(BG, dtype=jnp.int32) * N)[:, None]
    outs = []
    for g in range(G):
        bsl = slice(g * BG, (g + 1) * BG)
        nf_g = node_features[bsl]
        A_g, Bm_g = _tc1(nf_g, Wa, Wb, bn=2000)
        es_g = (edge_src[bsl] + offs).reshape(BG * E)
        ed_g = (edge_dst[bsl] + offs).reshape(BG * E)
        hsum_g = _sc1(A_g.reshape(BG * N, H), Bm_g.reshape(BG * N, H),
                      es_g, ed_g)
        msg_g = _tc2(hsum_g.reshape(BG, E, H), edge_features[bsl], Wc,
                     bm1.reshape(1, H), Wm2, bm2.reshape(1, H), be=2000)
        agg_g = _sc2(msg_g, edge_dst[bsl].reshape(BG * E), N)
        outs.append(_tc3(nf_g, agg_g, Wua, Wub,
                         bu1.reshape(1, H), Wu2, bu2.reshape(1, H), bn=2000))
    return jnp.concatenate(outs, axis=0)


# speedup vs baseline: 1.0399x; 1.0053x over previous
import functools

import jax
import jax.numpy as jnp
from jax import lax
from jax.experimental import pallas as pl
from jax.experimental.pallas import tpu as pltpu
from jax.experimental.pallas import tpu_sc as plsc

NC = 2
NS = 16
NW = NC * NS


def _tc1_body(nf_ref, wa_ref, wb_ref, a_ref, b_ref):
    x = nf_ref[0]
    a_ref[0] = jnp.dot(x, wa_ref[...], preferred_element_type=jnp.float32)
    b_ref[0] = jnp.dot(x, wb_ref[...], preferred_element_type=jnp.float32)


def _tc1(nf, Wa, Wb, bn):
    B, N, D = nf.shape
    H = Wa.shape[1]
    grid = (B, N // bn)
    return pl.pallas_call(
        _tc1_body,
        grid=grid,
        in_specs=[
            pl.BlockSpec((1, bn, D), lambda b, i: (b, i, 0)),
            pl.BlockSpec((D, H), lambda b, i: (0, 0)),
            pl.BlockSpec((D, H), lambda b, i: (0, 0)),
        ],
        out_specs=[
            pl.BlockSpec((1, bn, H), lambda b, i: (b, i, 0)),
            pl.BlockSpec((1, bn, H), lambda b, i: (b, i, 0)),
        ],
        out_shape=[
            jax.ShapeDtypeStruct((B, N, H), jnp.float32),
            jax.ShapeDtypeStruct((B, N, H), jnp.float32),
        ],
    )(nf, Wa, Wb)


def _sc1_body(K, EPW, H, a_hbm, b_hbm, es_hbm, ed_hbm, out_hbm,
              idxa, idxb, bufa0, bufb0, bufa1, bufb1,
              sa0, sb0, sa1, sb1):
    c = lax.axis_index("c")
    s = lax.axis_index("s")
    wid = s * NC + c
    ebase = wid * EPW
    C = EPW // K
    bufa = [bufa0, bufa1]
    bufb = [bufb0, bufb1]
    sa = [sa0, sa1]
    sb = [sb0, sb1]

    pltpu.sync_copy(es_hbm.at[pl.ds(ebase, EPW)], idxa)
    pltpu.sync_copy(ed_hbm.at[pl.ds(ebase, EPW)], idxb)

    def start(i, d):
        off = i * K
        pltpu.async_copy(a_hbm.at[idxa.at[pl.ds(off, K)]], bufa[d], sa[d])
        pltpu.async_copy(b_hbm.at[idxb.at[pl.ds(off, K)]], bufb[d], sb[d])

    def finish(i, d):
        off = i * K
        pltpu.make_async_copy(a_hbm.at[idxa.at[pl.ds(off, K)]], bufa[d],
                              sa[d]).wait()
        pltpu.make_async_copy(b_hbm.at[idxb.at[pl.ds(off, K)]], bufb[d],
                              sb[d]).wait()

        def row(r, cc):
            for j in range(H // 16):
                sl = pl.ds(j * 16, 16)
                plsc.addupdate(bufa[d].at[r, sl], bufb[d][r, sl])
            return cc

        lax.fori_loop(0, K, row, 0)
        pltpu.sync_copy(bufa[d], out_hbm.at[pl.ds(ebase + i * K, K)])

    start(0, 0)

    def pair(g, cc):
        i = g * 2
        start(i + 1, 1)
        finish(i, 0)

        @pl.when(i + 2 < C)
        def _():
            start(i + 2, 0)

        finish(i + 1, 1)
        return cc

    lax.fori_loop(0, C // 2, pair, 0)
    if C % 2 == 1:
        finish(C - 1, 0)


def _sc1(a_flat, b_flat, es_flat, ed_flat, K=40):
    BE = es_flat.shape[0]
    H = a_flat.shape[1]
    EPW = BE // NW
    mesh = plsc.VectorSubcoreMesh(core_axis_name="c", subcore_axis_name="s")
    fn = pl.kernel(
        functools.partial(_sc1_body, K, EPW, H),
        out_type=jax.ShapeDtypeStruct((BE, H), jnp.float32),
        mesh=mesh,
        scratch_types=(
            [pltpu.VMEM((EPW,), jnp.int32)] * 2
            + [pltpu.VMEM((K, H), jnp.float32)] * 4
            + [pltpu.SemaphoreType.DMA] * 4
        ),
    )
    return fn(a_flat, b_flat, es_flat, ed_flat)


def _tc2_body(hs_ref, ef_ref, wc_ref, bm1_ref, wm2_ref, bm2_ref, msg_ref):
    h = hs_ref[0] + jnp.dot(ef_ref[0], wc_ref[...],
                            preferred_element_type=jnp.float32) + bm1_ref[...]
    h = jnp.maximum(h, 0.0)
    m = jnp.dot(h, wm2_ref[...], preferred_element_type=jnp.float32) + bm2_ref[...]
    msg_ref[0] = jnp.maximum(m, 0.0)


def _tc2(hsum, ef, Wc, bm1, Wm2, bm2, be):
    B, E, H = hsum.shape
    DE = ef.shape[2]
    grid = (B, E // be)
    return pl.pallas_call(
        _tc2_body,
        grid=grid,
        in_specs=[
            pl.BlockSpec((1, be, H), lambda b, i: (b, i, 0)),
            pl.BlockSpec((1, be, DE), lambda b, i: (b, i, 0)),
            pl.BlockSpec((DE, H), lambda b, i: (0, 0)),
            pl.BlockSpec((1, H), lambda b, i: (0, 0)),
            pl.BlockSpec((H, H), lambda b, i: (0, 0)),
            pl.BlockSpec((1, H), lambda b, i: (0, 0)),
        ],
        out_specs=pl.BlockSpec((1, be, H), lambda b, i: (b, i, 0)),
        out_shape=jax.ShapeDtypeStruct((B, E, H), jnp.float32),
    )(hsum, ef, Wc, bm1, Wm2, bm2)


def _sc2_body(B, E, N, H, K, msg_hbm, edr_hbm, agg_hbm,
              idx, buf0, buf1, shared, sm0, sm1):
    HC = H // NC
    RZ = K
    NCH = N // RZ
    NT = (NCH + NS - 1) // NS
    EPT = E // NS
    CB = EPT // K
    c = lax.axis_index("c")
    s = lax.axis_index("s")
    col0 = c * HC
    buf = [buf0, buf1]
    sm = [sm0, sm1]

    def zrow(r, cc):
        for j in range(HC // 16):
            buf0[r, pl.ds(j * 16, 16)] = jnp.zeros((16,), jnp.float32)
        return cc

    def start(b, i, d):
        base = s * EPT + i * K
        pltpu.async_copy(msg_hbm.at[b, pl.ds(base, K), pl.ds(col0, HC)],
                         buf[d], sm[d])

    def finish(b, i, d):
        base = s * EPT + i * K
        pltpu.make_async_copy(msg_hbm.at[b, pl.ds(base, K), pl.ds(col0, HC)],
                              buf[d], sm[d]).wait()
        pltpu.sync_copy(buf[d], shared.at[idx.at[pl.ds(i * K, K)]], add=True)

    for b in range(B):
        pltpu.sync_copy(edr_hbm.at[pl.ds(b * E + s * EPT, EPT)], idx)
        lax.fori_loop(0, RZ, zrow, 0)
        for t in range(NT):
            j = s + t * NS

            @pl.when(j < NCH)
            def _():
                pltpu.sync_copy(buf0, shared.at[pl.ds(j * RZ, RZ)])

        plsc.subcore_barrier()

        start(b, 0, 0)

        def pair(g, cc):
            i = g * 2
            start(b, i + 1, 1)
            finish(b, i, 0)

            @pl.when(i + 2 < CB)
            def _():
                start(b, i + 2, 0)

            finish(b, i + 1, 1)
            return cc

        lax.fori_loop(0, CB // 2, pair, 0)
        plsc.subcore_barrier()

        for t in range(NT):
            j = s + t * NS

            @pl.when(j < NCH)
            def _():
                r0 = pl.multiple_of(j * RZ, RZ)
                pltpu.sync_copy(shared.at[pl.ds(r0, RZ)], buf0)
                pltpu.sync_copy(buf0, agg_hbm.at[b, pl.ds(r0, RZ), pl.ds(col0, HC)])

        plsc.subcore_barrier()


def _sc2(msg, ed_raw_flat, N, K=40):
    B, E, H = msg.shape
    EPT = E // NS
    mesh = plsc.VectorSubcoreMesh(core_axis_name="c", subcore_axis_name="s")
    fn = pl.kernel(
        functools.partial(_sc2_body, B, E, N, H, K),
        out_type=jax.ShapeDtypeStruct((B, N, H), jnp.float32),
        mesh=mesh,
        scratch_types=[
            pltpu.VMEM((EPT,), jnp.int32),
            pltpu.VMEM((K, H // NC), jnp.float32),
            pltpu.VMEM((K, H // NC), jnp.float32),
            pltpu.VMEM_SHARED((N, H // NC), jnp.float32),
            pltpu.SemaphoreType.DMA,
            pltpu.SemaphoreType.DMA,
        ],
    )
    return fn(msg, ed_raw_flat)


def _tc3_body(nf_ref, agg_ref, wua_ref, wub_ref, bu1_ref, wu2_ref, bu2_ref, out_ref):
    h = (jnp.dot(nf_ref[0], wua_ref[...], preferred_element_type=jnp.float32)
         + jnp.dot(agg_ref[0], wub_ref[...], preferred_element_type=jnp.float32)
         + bu1_ref[...])
    h = jnp.maximum(h, 0.0)
    o = jnp.dot(h, wu2_ref[...], preferred_element_type=jnp.float32) + bu2_ref[...]
    out_ref[0] = jnp.maximum(o, 0.0)


def _tc3(nf, agg, Wua, Wub, bu1, Wu2, bu2, bn):
    B, N, D = nf.shape
    H = agg.shape[2]
    grid = (B, N // bn)
    return pl.pallas_call(
        _tc3_body,
        grid=grid,
        in_specs=[
            pl.BlockSpec((1, bn, D), lambda b, i: (b, i, 0)),
            pl.BlockSpec((1, bn, H), lambda b, i: (b, i, 0)),
            pl.BlockSpec((D, H), lambda b, i: (0, 0)),
            pl.BlockSpec((H, H), lambda b, i: (0, 0)),
            pl.BlockSpec((1, H), lambda b, i: (0, 0)),
            pl.BlockSpec((H, H), lambda b, i: (0, 0)),
            pl.BlockSpec((1, H), lambda b, i: (0, 0)),
        ],
        out_specs=pl.BlockSpec((1, bn, H), lambda b, i: (b, i, 0)),
        out_shape=jax.ShapeDtypeStruct((B, N, H), jnp.float32),
    )(nf, agg, Wua, Wub, bu1, Wu2, bu2)


def kernel(node_features, edge_features, edge_src, edge_dst,
           Wm1, bm1, Wm2, bm2, Wu1, bu1, Wu2, bu2):
    B, N, D = node_features.shape
    E = edge_src.shape[1]
    H = Wm2.shape[0]

    Wa = Wm1[:D]
    Wb = Wm1[D:2 * D]
    Wc = Wm1[2 * D:]
    Wua = Wu1[:D]
    Wub = Wu1[D:]

    A, Bm = _tc1(node_features, Wa, Wb, bn=2000)
    A2 = A.reshape(B * N, H)
    B2 = Bm.reshape(B * N, H)

    offs = (jnp.arange(B, dtype=jnp.int32) * N)[:, None]
    es_flat = (edge_src + offs).reshape(B * E)
    ed_flat = (edge_dst + offs).reshape(B * E)

    G = 2
    BG = B // G
    outs = []
    for g in range(G):
        sl = slice(g * BG * E, (g + 1) * BG * E)
        bsl = slice(g * BG, (g + 1) * BG)
        hsum_g = _sc1(A2, B2, es_flat[sl], ed_flat[sl])
        msg_g = _tc2(hsum_g.reshape(BG, E, H), edge_features[bsl], Wc,
                     bm1.reshape(1, H), Wm2, bm2.reshape(1, H), be=2000)
        agg_g = _sc2(msg_g, edge_dst[bsl].reshape(BG * E), N)
        outs.append(_tc3(node_features[bsl], agg_g, Wua, Wub,
                         bu1.reshape(1, H), Wu2, bu2.reshape(1, H), bn=2000))
    return jnp.concatenate(outs, axis=0)


# speedup vs baseline: 1.1844x; 1.1389x over previous
import functools

import jax
import jax.numpy as jnp
from jax import lax
from jax.experimental import pallas as pl
from jax.experimental.pallas import tpu as pltpu
from jax.experimental.pallas import tpu_sc as plsc

NC = 2
NS = 16
NW = NC * NS


def _tc1_body(nf_ref, wa_ref, wb_ref, a_ref, b_ref):
    x = nf_ref[0]

    def pack(m):
        u = lax.bitcast_convert_type(m, jnp.int32)
        hw = u.shape[1] // 2
        lo16 = lax.shift_right_logical(u[:, :hw] + 0x8000, 16)
        hi16 = jnp.bitwise_and(u[:, hw:] + 0x8000, jnp.int32(-65536))
        return jnp.bitwise_or(hi16, lo16)

    a_ref[0] = pack(jnp.dot(x, wa_ref[...], preferred_element_type=jnp.float32))
    b_ref[0] = pack(jnp.dot(x, wb_ref[...], preferred_element_type=jnp.float32))


def _tc1(nf, Wa, Wb, bn):
    B, N, D = nf.shape
    H = Wa.shape[1]
    grid = (B, N // bn)
    return pl.pallas_call(
        _tc1_body,
        grid=grid,
        in_specs=[
            pl.BlockSpec((1, bn, D), lambda b, i: (b, i, 0)),
            pl.BlockSpec((D, H), lambda b, i: (0, 0)),
            pl.BlockSpec((D, H), lambda b, i: (0, 0)),
        ],
        out_specs=[
            pl.BlockSpec((1, bn, H // 2), lambda b, i: (b, i, 0)),
            pl.BlockSpec((1, bn, H // 2), lambda b, i: (b, i, 0)),
        ],
        out_shape=[
            jax.ShapeDtypeStruct((B, N, H // 2), jnp.int32),
            jax.ShapeDtypeStruct((B, N, H // 2), jnp.int32),
        ],
    )(nf, Wa, Wb)


def _sc1_body(K, EPW, H, a_hbm, b_hbm, es_hbm, ed_hbm, outa_hbm, outb_hbm,
              idxa, idxb, bufa0, bufb0, bufa1, bufb1,
              sa0, sb0, sa1, sb1):
    c = lax.axis_index("c")
    s = lax.axis_index("s")
    wid = s * NC + c
    ebase = wid * EPW
    C = EPW // K
    bufa = [bufa0, bufa1]
    bufb = [bufb0, bufb1]
    sa = [sa0, sa1]
    sb = [sb0, sb1]

    pltpu.sync_copy(es_hbm.at[pl.ds(ebase, EPW)], idxa)
    pltpu.sync_copy(ed_hbm.at[pl.ds(ebase, EPW)], idxb)

    def start(i, d):
        off = i * K
        pltpu.async_copy(a_hbm.at[idxa.at[pl.ds(off, K)]], bufa[d], sa[d])
        pltpu.async_copy(b_hbm.at[idxb.at[pl.ds(off, K)]], bufb[d], sb[d])

    def finish(i, d):
        off = i * K
        pltpu.make_async_copy(a_hbm.at[idxa.at[pl.ds(off, K)]], bufa[d],
                              sa[d]).wait()
        pltpu.make_async_copy(b_hbm.at[idxb.at[pl.ds(off, K)]], bufb[d],
                              sb[d]).wait()
        pltpu.sync_copy(bufa[d], outa_hbm.at[pl.ds(ebase + i * K, K)])
        pltpu.sync_copy(bufb[d], outb_hbm.at[pl.ds(ebase + i * K, K)])

    start(0, 0)

    def pair(g, cc):
        i = g * 2
        start(i + 1, 1)
        finish(i, 0)

        @pl.when(i + 2 < C)
        def _():
            start(i + 2, 0)

        finish(i + 1, 1)
        return cc

    lax.fori_loop(0, C // 2, pair, 0)
    if C % 2 == 1:
        finish(C - 1, 0)


def _sc1(a_flat, b_flat, es_flat, ed_flat, K=80):
    BE = es_flat.shape[0]
    H = a_flat.shape[1]
    EPW = BE // NW
    mesh = plsc.VectorSubcoreMesh(core_axis_name="c", subcore_axis_name="s")
    fn = pl.kernel(
        functools.partial(_sc1_body, K, EPW, H),
        out_type=[
            jax.ShapeDtypeStruct((BE, H), jnp.int32),
            jax.ShapeDtypeStruct((BE, H), jnp.int32),
        ],
        mesh=mesh,
        scratch_types=(
            [pltpu.VMEM((EPW,), jnp.int32)] * 2
            + [pltpu.VMEM((K, H), jnp.int32)] * 4
            + [pltpu.SemaphoreType.DMA] * 4
        ),
    )
    return fn(a_flat, b_flat, es_flat, ed_flat)


def _tc2_body(as_ref, bd_ref, ef_ref, wc_ref, bm1_ref, wm2_ref, bm2_ref,
              msg_ref):
    def unpack(mi):
        lo = lax.bitcast_convert_type(lax.shift_left(mi, 16), jnp.float32)
        hi = lax.bitcast_convert_type(
            jnp.bitwise_and(mi, jnp.int32(-65536)), jnp.float32)
        return jnp.concatenate([lo, hi], axis=1)

    h = (unpack(as_ref[0]) + unpack(bd_ref[0])
         + jnp.dot(ef_ref[0], wc_ref[...],
                   preferred_element_type=jnp.float32) + bm1_ref[...])
    h = jnp.maximum(h, 0.0)
    m = jnp.dot(h, wm2_ref[...], preferred_element_type=jnp.float32) + bm2_ref[...]
    msg_ref[0] = jnp.maximum(m, 0.0)


def _tc2(asrc, bdst, ef, Wc, bm1, Wm2, bm2, be):
    B, E, H2 = asrc.shape
    H = H2 * 2
    DE = ef.shape[2]
    grid = (B, E // be)
    return pl.pallas_call(
        _tc2_body,
        grid=grid,
        in_specs=[
            pl.BlockSpec((1, be, H2), lambda b, i: (b, i, 0)),
            pl.BlockSpec((1, be, H2), lambda b, i: (b, i, 0)),
            pl.BlockSpec((1, be, DE), lambda b, i: (b, i, 0)),
            pl.BlockSpec((DE, H), lambda b, i: (0, 0)),
            pl.BlockSpec((1, H), lambda b, i: (0, 0)),
            pl.BlockSpec((H, H), lambda b, i: (0, 0)),
            pl.BlockSpec((1, H), lambda b, i: (0, 0)),
        ],
        out_specs=pl.BlockSpec((1, be, H), lambda b, i: (b, i, 0)),
        out_shape=jax.ShapeDtypeStruct((B, E, H), jnp.float32),
    )(asrc, bdst, ef, Wc, bm1, Wm2, bm2)


def _sc2_body(B, E, N, H, K, msg_hbm, edr_hbm, agg_hbm,
              idx, buf0, buf1, shared, sm0, sm1):
    HC = H // NC
    RZ = K
    NCH = N // RZ
    NT = (NCH + NS - 1) // NS
    EPT = E // NS
    CB = EPT // K
    c = lax.axis_index("c")
    s = lax.axis_index("s")
    col0 = c * HC
    buf = [buf0, buf1]
    sm = [sm0, sm1]

    def zrow(r, cc):
        for j in range(HC // 16):
            buf0[r, pl.ds(j * 16, 16)] = jnp.zeros((16,), jnp.float32)
        return cc

    def start(b, i, d):
        base = s * EPT + i * K
        pltpu.async_copy(msg_hbm.at[b, pl.ds(base, K), pl.ds(col0, HC)],
                         buf[d], sm[d])

    def finish(b, i, d):
        base = s * EPT + i * K
        pltpu.make_async_copy(msg_hbm.at[b, pl.ds(base, K), pl.ds(col0, HC)],
                              buf[d], sm[d]).wait()
        pltpu.sync_copy(buf[d], shared.at[idx.at[pl.ds(i * K, K)]], add=True)

    for b in range(B):
        pltpu.sync_copy(edr_hbm.at[pl.ds(b * E + s * EPT, EPT)], idx)
        lax.fori_loop(0, RZ, zrow, 0)
        for t in range(NT):
            j = s + t * NS

            @pl.when(j < NCH)
            def _():
                pltpu.sync_copy(buf0, shared.at[pl.ds(j * RZ, RZ)])

        plsc.subcore_barrier()

        start(b, 0, 0)

        def pair(g, cc):
            i = g * 2
            start(b, i + 1, 1)
            finish(b, i, 0)

            @pl.when(i + 2 < CB)
            def _():
                start(b, i + 2, 0)

            finish(b, i + 1, 1)
            return cc

        lax.fori_loop(0, CB // 2, pair, 0)
        plsc.subcore_barrier()

        for t in range(NT):
            j = s + t * NS

            @pl.when(j < NCH)
            def _():
                r0 = pl.multiple_of(j * RZ, RZ)
                pltpu.sync_copy(shared.at[pl.ds(r0, RZ)], buf0)
                pltpu.sync_copy(buf0, agg_hbm.at[b, pl.ds(r0, RZ), pl.ds(col0, HC)])

        plsc.subcore_barrier()


def _sc2(msg, ed_raw_flat, N, K=40):
    B, E, H = msg.shape
    EPT = E // NS
    mesh = plsc.VectorSubcoreMesh(core_axis_name="c", subcore_axis_name="s")
    fn = pl.kernel(
        functools.partial(_sc2_body, B, E, N, H, K),
        out_type=jax.ShapeDtypeStruct((B, N, H), jnp.float32),
        mesh=mesh,
        scratch_types=[
            pltpu.VMEM((EPT,), jnp.int32),
            pltpu.VMEM((K, H // NC), jnp.float32),
            pltpu.VMEM((K, H // NC), jnp.float32),
            pltpu.VMEM_SHARED((N, H // NC), jnp.float32),
            pltpu.SemaphoreType.DMA,
            pltpu.SemaphoreType.DMA,
        ],
    )
    return fn(msg, ed_raw_flat)


def _tc3_body(nf_ref, agg_ref, wua_ref, wub_ref, bu1_ref, wu2_ref, bu2_ref, out_ref):
    h = (jnp.dot(nf_ref[0], wua_ref[...], preferred_element_type=jnp.float32)
         + jnp.dot(agg_ref[0], wub_ref[...], preferred_element_type=jnp.float32)
         + bu1_ref[...])
    h = jnp.maximum(h, 0.0)
    o = jnp.dot(h, wu2_ref[...], preferred_element_type=jnp.float32) + bu2_ref[...]
    out_ref[0] = jnp.maximum(o, 0.0)


def _tc3(nf, agg, Wua, Wub, bu1, Wu2, bu2, bn):
    B, N, D = nf.shape
    H = agg.shape[2]
    grid = (B, N // bn)
    return pl.pallas_call(
        _tc3_body,
        grid=grid,
        in_specs=[
            pl.BlockSpec((1, bn, D), lambda b, i: (b, i, 0)),
            pl.BlockSpec((1, bn, H), lambda b, i: (b, i, 0)),
            pl.BlockSpec((D, H), lambda b, i: (0, 0)),
            pl.BlockSpec((H, H), lambda b, i: (0, 0)),
            pl.BlockSpec((1, H), lambda b, i: (0, 0)),
            pl.BlockSpec((H, H), lambda b, i: (0, 0)),
            pl.BlockSpec((1, H), lambda b, i: (0, 0)),
        ],
        out_specs=pl.BlockSpec((1, bn, H), lambda b, i: (b, i, 0)),
        out_shape=jax.ShapeDtypeStruct((B, N, H), jnp.float32),
    )(nf, agg, Wua, Wub, bu1, Wu2, bu2)


def kernel(node_features, edge_features, edge_src, edge_dst,
           Wm1, bm1, Wm2, bm2, Wu1, bu1, Wu2, bu2):
    B, N, D = node_features.shape
    E = edge_src.shape[1]
    H = Wm2.shape[0]

    Wa = Wm1[:D]
    Wb = Wm1[D:2 * D]
    Wc = Wm1[2 * D:]
    Wua = Wu1[:D]
    Wub = Wu1[D:]

    A, Bm = _tc1(node_features, Wa, Wb, bn=2000)
    A2 = A.reshape(B * N, H // 2)
    B2 = Bm.reshape(B * N, H // 2)

    offs = (jnp.arange(B, dtype=jnp.int32) * N)[:, None]
    es_flat = (edge_src + offs).reshape(B * E)
    ed_flat = (edge_dst + offs).reshape(B * E)

    G = 2
    BG = B // G
    outs = []
    for g in range(G):
        sl = slice(g * BG * E, (g + 1) * BG * E)
        bsl = slice(g * BG, (g + 1) * BG)
        as_g, bd_g = _sc1(A2, B2, es_flat[sl], ed_flat[sl])
        msg_g = _tc2(as_g.reshape(BG, E, H // 2), bd_g.reshape(BG, E, H // 2),
                     edge_features[bsl], Wc,
                     bm1.reshape(1, H), Wm2, bm2.reshape(1, H), be=2000)
        agg_g = _sc2(msg_g, edge_dst[bsl].reshape(BG * E), N)
        outs.append(_tc3(node_features[bsl], agg_g, Wua, Wub,
                         bu1.reshape(1, H), Wu2, bu2.reshape(1, H), bn=2000))
    return jnp.concatenate(outs, axis=0)
